# baseline (speedup 1.0000x reference)
import functools

import jax
import jax.numpy as jnp
from jax import lax
from jax.experimental import pallas as pl
from jax.experimental.pallas import tpu as pltpu
from jax.experimental.pallas import tpu_sc as plsc

HID = 64
D = 2 * HID
NSUB = 16
CB = 128
F32 = jnp.float32
I32 = jnp.int32


def _rup(x, m):
    return (x + m - 1) // m * m


@functools.lru_cache(maxsize=None)
def _edge_agg_call(e_pad, two_n_src, n_dst_pad):
    m = e_pad // NSUB
    n_chunks = m // CB
    zrows = n_dst_pad // NSUB
    zf = zrows // CB
    zr = zrows % CB

    mesh = plsc.VectorSubcoreMesh(core_axis_name="c", subcore_axis_name="s")

    def body(xsrc, srci, dsti, eav, wb, out,
             idx0, idx1, ea_v, dst0, dst1,
             rows0, rows1, msg0, msg1, wb_v, agg,
             gsem0, gsem1, ssem0, ssem1):
        c = lax.axis_index("c")
        s = lax.axis_index("s")
        zero16 = jnp.zeros((16,), F32)
        rows = (rows0, rows1)
        idxb = (idx0, idx1)
        dstb = (dst0, dst1)
        msgb = (msg0, msg1)
        gsem = (gsem0, gsem1)
        ssem = (ssem0, ssem1)
        row0 = s * n_chunks

        pltpu.sync_copy(wb.at[c], wb_v)

        pltpu.sync_copy(srci.at[c].at[row0], idx0)
        pltpu.sync_copy(srci.at[c].at[row0 + 1], idx1)
        pltpu.async_copy(xsrc.at[idx0], rows0, gsem0)
        pltpu.async_copy(xsrc.at[idx1], rows1, gsem1)

        for r in range(CB):
            for j in range(HID // 16):
                msg0[r, pl.ds(j * 16, 16)] = zero16

        def zfill(k, _):
            pltpu.sync_copy(msg0, agg.at[pl.ds(s * zrows + k * CB, CB)])
            return 0
        if zf:
            lax.fori_loop(0, zf, zfill, 0)
        if zr:
            pltpu.sync_copy(msg0.at[pl.ds(0, zr)],
                            agg.at[pl.ds(s * zrows + zf * CB, zr)])
        plsc.subcore_barrier()

        wv = [wb_v[0, pl.ds(j * 16, 16)] for j in range(HID // 16)]
        bv = [wb_v[1, pl.ds(j * 16, 16)] for j in range(HID // 16)]

        def step(i, _):
            for b in range(2):
                t = 2 * i + b
                rb = rows[b]
                mb = msgb[b]
                pltpu.make_async_copy(xsrc.at[idxb[b]], rb, gsem[b]).wait()
                pltpu.sync_copy(eav.at[row0 + t], ea_v)

                @pl.when(t >= 2)
                def _():
                    pltpu.make_async_copy(mb, agg.at[dstb[b]],
                                          ssem[b]).wait()

                for g in range(CB // 16):
                    ea16 = ea_v[pl.ds(g * 16, 16)]
                    for r in range(16):
                        eab = jnp.broadcast_to(ea16[r], (16,))
                        row = g * 16 + r
                        for j in range(HID // 16):
                            sl = pl.ds(j * 16, 16)
                            mb[row, sl] = jnp.maximum(
                                rb[row, sl] + (eab * wv[j] + bv[j]), 0.0)

                @pl.when(t < n_chunks - 2)
                def _():
                    pltpu.sync_copy(srci.at[c].at[row0 + t + 2], idxb[b])
                    pltpu.async_copy(xsrc.at[idxb[b]], rb, gsem[b])

                pltpu.sync_copy(dsti.at[row0 + t], dstb[b])
                pltpu.async_copy(mb, agg.at[dstb[b]], ssem[b], add=True)
            return 0

        lax.fori_loop(0, n_chunks // 2, step, 0)

        for b in range(2):
            pltpu.make_async_copy(msgb[b], agg.at[dstb[b]], ssem[b]).wait()
        plsc.subcore_barrier()

        pltpu.sync_copy(agg.at[pl.ds(s * zrows, zrows)],
                        out.at[pl.ds(c * n_dst_pad + s * zrows, zrows)])

    return pl.kernel(
        body,
        out_type=jax.ShapeDtypeStruct((2 * n_dst_pad, HID), F32),
        mesh=mesh,
        compiler_params=pltpu.CompilerParams(use_tc_tiling_on_sc=False),
        scratch_types=[
            pltpu.VMEM((CB,), I32),
            pltpu.VMEM((CB,), I32),
            pltpu.VMEM((CB,), F32),
            pltpu.VMEM((CB,), I32),
            pltpu.VMEM((CB,), I32),
            pltpu.VMEM((CB, HID), F32),
            pltpu.VMEM((CB, HID), F32),
            pltpu.VMEM((CB, HID), F32),
            pltpu.VMEM((CB, HID), F32),
            pltpu.VMEM((2, HID), F32),
            pltpu.VMEM_SHARED((n_dst_pad, HID), F32),
            pltpu.SemaphoreType.DMA,
            pltpu.SemaphoreType.DMA,
            pltpu.SemaphoreType.DMA,
            pltpu.SemaphoreType.DMA,
        ],
        name="edge_agg_sc",
    )


def _edge_agg(x_src, ei, ea, edge_p, n_dst):
    e = ei.shape[1]
    n_src = x_src.shape[0]
    e_pad = _rup(e, NSUB * CB * 2)
    n_dst_pad = _rup(n_dst + 1, NSUB * 8)
    ntot = e_pad // CB
    src0 = jnp.pad(ei[0], (0, e_pad - e))
    srci = jnp.stack([src0, src0 + n_src]).reshape(2, ntot, CB)
    dsti = jnp.pad(ei[1], (0, e_pad - e),
                   constant_values=n_dst).reshape(ntot, CB)
    eav = jnp.pad(ea[:, 0], (0, e_pad - e)).reshape(ntot, CB)
    xsrc = jnp.concatenate([x_src[:, :HID], x_src[:, HID:]], axis=0)
    wb = jnp.stack([edge_p["W"][0].reshape(2, HID),
                    edge_p["b"].reshape(2, HID)], axis=1)
    out = _edge_agg_call(e_pad, 2 * n_src, n_dst_pad)(
        xsrc, srci, dsti, eav, wb)
    return out.reshape(2, n_dst_pad, HID)


def _enc_body(x_ref, w1_ref, b1_ref, g_ref, be_ref, w2_ref, b2_ref, o_ref):
    h = jnp.dot(x_ref[...], w1_ref[...], preferred_element_type=F32, precision=lax.Precision.HIGHEST)
    h = h + b1_ref[...]
    mu = jnp.mean(h, axis=0, keepdims=True)
    var = jnp.mean((h - mu) ** 2, axis=0, keepdims=True)
    h = g_ref[...] * (h - mu) / jnp.sqrt(var + 1e-5) + be_ref[...]
    h = jnp.maximum(h, 0.0)
    o_ref[...] = jnp.dot(h, w2_ref[...], preferred_element_type=F32, precision=lax.Precision.HIGHEST) \
        + b2_ref[...]


def _encode(x, p):
    n = x.shape[0]
    return pl.pallas_call(
        _enc_body,
        out_shape=jax.ShapeDtypeStruct((n, D), F32),
        name="encoder_tc",
    )(x, p[0]["W"], p[0]["b"].reshape(1, HID), p[0]["gamma"].reshape(1, HID),
      p[0]["beta"].reshape(1, HID), p[1]["W"], p[1]["b"].reshape(1, D))


def _gine_mlp(h, mlp, eps_w1, eps_b1):
    t = jnp.dot(h, mlp[0]["W"], preferred_element_type=F32, precision=lax.Precision.HIGHEST) \
        + mlp[0]["b"].reshape(1, HID)
    t = jnp.maximum(t, 0.0)
    return jnp.dot(t, mlp[1]["W"], preferred_element_type=F32, precision=lax.Precision.HIGHEST) \
        + mlp[1]["b"].reshape(1, HID)


def _post_body(x_ref, agga_ref, aggb_ref, eps_ref,
               wa1_ref, ba1_ref, wa2_ref, ba2_ref,
               wb1_ref, bb1_ref, wb2_ref, bb2_ref,
               h2_ref, xn_ref):
    x = x_ref[...]
    agga = jnp.concatenate([agga_ref[0], agga_ref[1]], axis=1)
    aggb = jnp.concatenate([aggb_ref[0], aggb_ref[1]], axis=1)
    ha_in = (1.0 + eps_ref[0, 0]) * x + agga
    hb_in = (1.0 + eps_ref[0, 1]) * x + aggb

    ta = jnp.maximum(jnp.dot(ha_in, wa1_ref[...],
                             preferred_element_type=F32, precision=lax.Precision.HIGHEST) + ba1_ref[...], 0.0)
    ha = jnp.dot(ta, wa2_ref[...], preferred_element_type=F32, precision=lax.Precision.HIGHEST) + ba2_ref[...]
    tb = jnp.maximum(jnp.dot(hb_in, wb1_ref[...],
                             preferred_element_type=F32, precision=lax.Precision.HIGHEST) + bb1_ref[...], 0.0)
    hb = jnp.dot(tb, wb2_ref[...], preferred_element_type=F32, precision=lax.Precision.HIGHEST) + bb2_ref[...]

    h2 = jnp.concatenate([ha, hb], axis=1)
    h2_ref[...] = h2
    xn_ref[...] = (jnp.maximum(h2, 0.0) + x) * 0.5


def _post(x, agg_a, agg_b, conv_a, conv_b, blk):
    n = x.shape[0]
    grid = n // blk
    n_pad = agg_a.shape[1]
    full = lambda shape: pl.BlockSpec(shape, lambda i: (0,) * len(shape))
    specs = [
        pl.BlockSpec((blk, D), lambda i: (i, 0)),
        pl.BlockSpec((2, blk, HID), lambda i: (0, i, 0)),
        pl.BlockSpec((2, blk, HID), lambda i: (0, i, 0)),
        full((1, 2)),
        full((D, HID)), full((1, HID)), full((HID, HID)), full((1, HID)),
        full((D, HID)), full((1, HID)), full((HID, HID)), full((1, HID)),
    ]
    eps_pair = jnp.stack([conv_a["eps"], conv_b["eps"]]).reshape(1, 2)
    ma, mb = conv_a["mlp"], conv_b["mlp"]
    return pl.pallas_call(
        _post_body,
        grid=(grid,),
        in_specs=specs,
        out_specs=[pl.BlockSpec((blk, D), lambda i: (i, 0))] * 2,
        out_shape=[jax.ShapeDtypeStruct((n, D), F32)] * 2,
        name="post_mlp_tc",
    )(x, agg_a, agg_b, eps_pair,
      ma[0]["W"], ma[0]["b"].reshape(1, HID), ma[1]["W"],
      ma[1]["b"].reshape(1, HID),
      mb[0]["W"], mb[0]["b"].reshape(1, HID), mb[1]["W"],
      mb[1]["b"].reshape(1, HID))


def _pred_body(h0_ref, h1_ref, w01_ref, b01_ref, w02_ref, b02_ref,
               w11_ref, b11_ref, w12_ref, b12_ref, o_ref):
    def head(h, w1, b1, w2col, b2):
        t = jnp.maximum(jnp.dot(h, w1, preferred_element_type=F32, precision=lax.Precision.HIGHEST) + b1, 0.0)
        return jnp.dot(t, w2col, preferred_element_type=F32,
                       precision=lax.Precision.HIGHEST) + b2
    c0 = head(h0_ref[...], w01_ref[...], b01_ref[...], w02_ref[...],
              b02_ref[0, 0])
    c1 = head(h1_ref[...], w11_ref[...], b11_ref[...], w12_ref[...],
              b12_ref[0, 0])
    o_ref[...] = jnp.concatenate([c0, c1], axis=1)


def _pred(h0, h1, p0, p1, blk):
    n = h0.shape[0]
    grid = n // blk
    full = lambda shape: pl.BlockSpec(shape, lambda i: (0,) * len(shape))
    specs = [
        pl.BlockSpec((blk, D), lambda i: (i, 0)),
        pl.BlockSpec((blk, D), lambda i: (i, 0)),
        full((D, HID)), full((1, HID)), full((HID, 1)), full((1, 1)),
        full((D, HID)), full((1, HID)), full((HID, 1)), full((1, 1)),
    ]
    return pl.pallas_call(
        _pred_body,
        grid=(grid,),
        in_specs=specs,
        out_specs=pl.BlockSpec((blk, 2), lambda i: (i, 0)),
        out_shape=jax.ShapeDtypeStruct((n, 2), F32),
        name="pred_tc",
    )(h0, h1,
      p0[0]["W"], p0[0]["b"].reshape(1, HID), p0[1]["W"].reshape(HID, 1),
      p0[1]["b"].reshape(1, 1),
      p1[0]["W"], p1[0]["b"].reshape(1, HID), p1[1]["W"].reshape(HID, 1),
      p1[1]["b"].reshape(1, 1))


def kernel(x_cons, x_vals, x_obj, ei_c2v, ei_v2c, ei_v2o, ei_o2v, ei_c2o,
           ei_o2c, ea_c2v, ea_v2c, ea_v2o, ea_o2v, ea_c2o, ea_o2c, params):
    nc, nv, no = x_cons.shape[0], x_vals.shape[0], x_obj.shape[0]
    x = {
        "cons": _encode(x_cons, params["enc_cons"]),
        "vals": _encode(x_vals, params["enc_vals"]),
        "obj": _encode(x_obj, params["enc_obj"]),
    }
    ei = {"c2v": ei_c2v, "v2c": ei_v2c, "v2o": ei_v2o, "o2v": ei_o2v,
          "c2o": ei_c2o, "o2c": ei_o2c}
    ea = {"c2v": ea_c2v, "v2c": ea_v2c, "v2o": ea_v2o, "o2v": ea_o2v,
          "c2o": ea_c2o, "o2c": ea_o2c}
    topo = {"c2v": ("cons", "vals", nv), "v2c": ("vals", "cons", nc),
            "v2o": ("vals", "obj", no), "o2v": ("obj", "vals", nv),
            "c2o": ("cons", "obj", no), "o2c": ("obj", "cons", nc)}
    blk = {"vals": 2000, "cons": 2000, "obj": 1000}

    hiddens = []
    for i in range(len(params["gcns"])):
        L = params["gcns"][i]
        agg = {}
        for name in topo:
            src, _, n_dst = topo[name]
            agg[name] = _edge_agg(x[src], ei[name], ea[name],
                                  L[name]["edge"], n_dst)
        h2_vals, xn_vals = _post(x["vals"], agg["c2v"], agg["o2v"],
                                 L["c2v"], L["o2v"], blk["vals"])
        h2_cons, xn_cons = _post(x["cons"], agg["v2c"], agg["o2c"],
                                 L["v2c"], L["o2c"], blk["cons"])
        h2_obj, xn_obj = _post(x["obj"], agg["v2o"], agg["c2o"],
                               L["v2o"], L["c2o"], blk["obj"])
        hiddens.append((h2_cons, h2_vals))
        x = {"cons": xn_cons, "vals": xn_vals, "obj": xn_obj}

    vals = _pred(hiddens[0][1], hiddens[1][1], params["pred_vals"][0],
                 params["pred_vals"][1], blk["vals"])
    cons = _pred(hiddens[0][0], hiddens[1][0], params["pred_cons"][0],
                 params["pred_cons"][1], blk["cons"])
    return vals, cons


# speedup vs baseline: 1.0311x; 1.0311x over previous
import functools

import jax
import jax.numpy as jnp
from jax import lax
from jax.experimental import pallas as pl
from jax.experimental.pallas import tpu as pltpu
from jax.experimental.pallas import tpu_sc as plsc

HID = 64
D = 2 * HID
NSUB = 16
CB = 128
F32 = jnp.float32
I32 = jnp.int32


def _rup(x, m):
    return (x + m - 1) // m * m


@functools.lru_cache(maxsize=None)
def _edge_agg_call(e_pad, two_n_src, n_dst_pad):
    m = e_pad // NSUB
    n_chunks = m // CB
    zrows = n_dst_pad // NSUB
    zf = zrows // CB
    zr = zrows % CB

    mesh = plsc.VectorSubcoreMesh(core_axis_name="c", subcore_axis_name="s")

    def body(xsrc, srci, dsti, eav, wb, out,
             idx0, idx1, ea_v, dst0, dst1,
             rows0, rows1, msg0, msg1, wb_v, agg,
             gsem0, gsem1, ssem0, ssem1):
        c = lax.axis_index("c")
        s = lax.axis_index("s")
        zero16 = jnp.zeros((16,), F32)
        rows = (rows0, rows1)
        idxb = (idx0, idx1)
        dstb = (dst0, dst1)
        msgb = (msg0, msg1)
        gsem = (gsem0, gsem1)
        ssem = (ssem0, ssem1)
        row0 = s * n_chunks

        pltpu.sync_copy(wb.at[c], wb_v)

        pltpu.sync_copy(srci.at[c].at[row0], idx0)
        pltpu.sync_copy(srci.at[c].at[row0 + 1], idx1)
        pltpu.async_copy(xsrc.at[idx0], rows0, gsem0)
        pltpu.async_copy(xsrc.at[idx1], rows1, gsem1)

        for r in range(CB):
            for j in range(HID // 16):
                msg0[r, pl.ds(j * 16, 16)] = zero16

        def zfill(k, _):
            pltpu.sync_copy(msg0, agg.at[pl.ds(s * zrows + k * CB, CB)])
            return 0
        if zf:
            lax.fori_loop(0, zf, zfill, 0)
        if zr:
            pltpu.sync_copy(msg0.at[pl.ds(0, zr)],
                            agg.at[pl.ds(s * zrows + zf * CB, zr)])
        plsc.subcore_barrier()

        wv = [wb_v[0, pl.ds(j * 16, 16)] for j in range(HID // 16)]
        bv = [wb_v[1, pl.ds(j * 16, 16)] for j in range(HID // 16)]

        def step(i, _):
            for b in range(2):
                t = 2 * i + b
                rb = rows[b]
                mb = msgb[b]
                pltpu.make_async_copy(xsrc.at[idxb[b]], rb, gsem[b]).wait()
                pltpu.sync_copy(eav.at[row0 + t], ea_v)

                @pl.when(t >= 2)
                def _():
                    pltpu.make_async_copy(mb, agg.at[dstb[b]],
                                          ssem[b]).wait()

                for g in range(CB // 16):
                    ea16 = ea_v[pl.ds(g * 16, 16)]
                    for r in range(16):
                        eab = jnp.broadcast_to(ea16[r], (16,))
                        row = g * 16 + r
                        for j in range(HID // 16):
                            sl = pl.ds(j * 16, 16)
                            mb[row, sl] = jnp.maximum(
                                rb[row, sl] + eab * wv[j] + bv[j], 0.0)

                @pl.when(t < n_chunks - 2)
                def _():
                    pltpu.sync_copy(srci.at[c].at[row0 + t + 2], idxb[b])
                    pltpu.async_copy(xsrc.at[idxb[b]], rb, gsem[b])

                pltpu.sync_copy(dsti.at[row0 + t], dstb[b])
                pltpu.async_copy(mb, agg.at[dstb[b]], ssem[b], add=True)
            return 0

        lax.fori_loop(0, n_chunks // 2, step, 0)

        for b in range(2):
            pltpu.make_async_copy(msgb[b], agg.at[dstb[b]], ssem[b]).wait()
        plsc.subcore_barrier()

        pltpu.sync_copy(agg.at[pl.ds(s * zrows, zrows)],
                        out.at[pl.ds(c * n_dst_pad + s * zrows, zrows)])

    return pl.kernel(
        body,
        out_type=jax.ShapeDtypeStruct((2 * n_dst_pad, HID), F32),
        mesh=mesh,
        compiler_params=pltpu.CompilerParams(use_tc_tiling_on_sc=False),
        scratch_types=[
            pltpu.VMEM((CB,), I32),
            pltpu.VMEM((CB,), I32),
            pltpu.VMEM((CB,), F32),
            pltpu.VMEM((CB,), I32),
            pltpu.VMEM((CB,), I32),
            pltpu.VMEM((CB, HID), F32),
            pltpu.VMEM((CB, HID), F32),
            pltpu.VMEM((CB, HID), F32),
            pltpu.VMEM((CB, HID), F32),
            pltpu.VMEM((2, HID), F32),
            pltpu.VMEM_SHARED((n_dst_pad, HID), F32),
            pltpu.SemaphoreType.DMA,
            pltpu.SemaphoreType.DMA,
            pltpu.SemaphoreType.DMA,
            pltpu.SemaphoreType.DMA,
        ],
        name="edge_agg_sc",
    )


def _edge_agg(x_src, ei, ea, edge_p, n_dst):
    e = ei.shape[1]
    n_src = x_src.shape[0]
    e_pad = _rup(e, NSUB * CB * 2)
    n_dst_pad = _rup(n_dst + 1, NSUB * 8)
    ntot = e_pad // CB
    src0 = jnp.pad(ei[0], (0, e_pad - e))
    srci = jnp.stack([src0, src0 + n_src]).reshape(2, ntot, CB)
    dsti = jnp.pad(ei[1], (0, e_pad - e),
                   constant_values=n_dst).reshape(ntot, CB)
    eav = jnp.pad(ea[:, 0], (0, e_pad - e)).reshape(ntot, CB)
    xsrc = jnp.concatenate([x_src[:, :HID], x_src[:, HID:]], axis=0)
    wb = jnp.stack([edge_p["W"][0].reshape(2, HID),
                    edge_p["b"].reshape(2, HID)], axis=1)
    out = _edge_agg_call(e_pad, 2 * n_src, n_dst_pad)(
        xsrc, srci, dsti, eav, wb)
    return out.reshape(2, n_dst_pad, HID)


def _enc_body(x_ref, w1_ref, b1_ref, g_ref, be_ref, w2_ref, b2_ref, o_ref):
    h = jnp.dot(x_ref[...], w1_ref[...], preferred_element_type=F32)
    h = h + b1_ref[...]
    mu = jnp.mean(h, axis=0, keepdims=True)
    var = jnp.mean((h - mu) ** 2, axis=0, keepdims=True)
    h = g_ref[...] * (h - mu) / jnp.sqrt(var + 1e-5) + be_ref[...]
    h = jnp.maximum(h, 0.0)
    o_ref[...] = jnp.dot(h, w2_ref[...], preferred_element_type=F32) \
        + b2_ref[...]


def _encode(x, p):
    n = x.shape[0]
    return pl.pallas_call(
        _enc_body,
        out_shape=jax.ShapeDtypeStruct((n, D), F32),
        name="encoder_tc",
    )(x, p[0]["W"], p[0]["b"].reshape(1, HID), p[0]["gamma"].reshape(1, HID),
      p[0]["beta"].reshape(1, HID), p[1]["W"], p[1]["b"].reshape(1, D))


def _gine_mlp(h, mlp, eps_w1, eps_b1):
    t = jnp.dot(h, mlp[0]["W"], preferred_element_type=F32) \
        + mlp[0]["b"].reshape(1, HID)
    t = jnp.maximum(t, 0.0)
    return jnp.dot(t, mlp[1]["W"], preferred_element_type=F32) \
        + mlp[1]["b"].reshape(1, HID)


def _post_body(x_ref, agga_ref, aggb_ref, eps_ref,
               wa1_ref, ba1_ref, wa2_ref, ba2_ref,
               wb1_ref, bb1_ref, wb2_ref, bb2_ref,
               h2_ref, xn_ref):
    x = x_ref[...]
    agga = jnp.concatenate([agga_ref[0], agga_ref[1]], axis=1)
    aggb = jnp.concatenate([aggb_ref[0], aggb_ref[1]], axis=1)
    ha_in = (1.0 + eps_ref[0, 0]) * x + agga
    hb_in = (1.0 + eps_ref[0, 1]) * x + aggb

    ta = jnp.maximum(jnp.dot(ha_in, wa1_ref[...],
                             preferred_element_type=F32) + ba1_ref[...], 0.0)
    ha = jnp.dot(ta, wa2_ref[...], preferred_element_type=F32) + ba2_ref[...]
    tb = jnp.maximum(jnp.dot(hb_in, wb1_ref[...],
                             preferred_element_type=F32) + bb1_ref[...], 0.0)
    hb = jnp.dot(tb, wb2_ref[...], preferred_element_type=F32) + bb2_ref[...]

    h2 = jnp.concatenate([ha, hb], axis=1)
    h2_ref[...] = h2
    xn_ref[...] = (jnp.maximum(h2, 0.0) + x) * 0.5


def _post(x, agg_a, agg_b, conv_a, conv_b, blk):
    n = x.shape[0]
    grid = n // blk
    n_pad = agg_a.shape[1]
    full = lambda shape: pl.BlockSpec(shape, lambda i: (0,) * len(shape))
    specs = [
        pl.BlockSpec((blk, D), lambda i: (i, 0)),
        pl.BlockSpec((2, blk, HID), lambda i: (0, i, 0)),
        pl.BlockSpec((2, blk, HID), lambda i: (0, i, 0)),
        full((1, 2)),
        full((D, HID)), full((1, HID)), full((HID, HID)), full((1, HID)),
        full((D, HID)), full((1, HID)), full((HID, HID)), full((1, HID)),
    ]
    eps_pair = jnp.stack([conv_a["eps"], conv_b["eps"]]).reshape(1, 2)
    ma, mb = conv_a["mlp"], conv_b["mlp"]
    return pl.pallas_call(
        _post_body,
        grid=(grid,),
        in_specs=specs,
        out_specs=[pl.BlockSpec((blk, D), lambda i: (i, 0))] * 2,
        out_shape=[jax.ShapeDtypeStruct((n, D), F32)] * 2,
        name="post_mlp_tc",
    )(x, agg_a, agg_b, eps_pair,
      ma[0]["W"], ma[0]["b"].reshape(1, HID), ma[1]["W"],
      ma[1]["b"].reshape(1, HID),
      mb[0]["W"], mb[0]["b"].reshape(1, HID), mb[1]["W"],
      mb[1]["b"].reshape(1, HID))


def _pred_body(h0_ref, h1_ref, w01_ref, b01_ref, w02_ref, b02_ref,
               w11_ref, b11_ref, w12_ref, b12_ref, o_ref):
    def head(h, w1, b1, w2col, b2):
        t = jnp.maximum(jnp.dot(h, w1, preferred_element_type=F32) + b1, 0.0)
        return jnp.dot(t, w2col, preferred_element_type=F32) + b2
    c0 = head(h0_ref[...], w01_ref[...], b01_ref[...], w02_ref[...],
              b02_ref[0, 0])
    c1 = head(h1_ref[...], w11_ref[...], b11_ref[...], w12_ref[...],
              b12_ref[0, 0])
    o_ref[...] = jnp.concatenate([c0, c1], axis=1)


def _pred(h0, h1, p0, p1, blk):
    n = h0.shape[0]
    grid = n // blk
    full = lambda shape: pl.BlockSpec(shape, lambda i: (0,) * len(shape))
    specs = [
        pl.BlockSpec((blk, D), lambda i: (i, 0)),
        pl.BlockSpec((blk, D), lambda i: (i, 0)),
        full((D, HID)), full((1, HID)), full((HID, 1)), full((1, 1)),
        full((D, HID)), full((1, HID)), full((HID, 1)), full((1, 1)),
    ]
    return pl.pallas_call(
        _pred_body,
        grid=(grid,),
        in_specs=specs,
        out_specs=pl.BlockSpec((blk, 2), lambda i: (i, 0)),
        out_shape=jax.ShapeDtypeStruct((n, 2), F32),
        name="pred_tc",
    )(h0, h1,
      p0[0]["W"], p0[0]["b"].reshape(1, HID), p0[1]["W"].reshape(HID, 1),
      p0[1]["b"].reshape(1, 1),
      p1[0]["W"], p1[0]["b"].reshape(1, HID), p1[1]["W"].reshape(HID, 1),
      p1[1]["b"].reshape(1, 1))


def kernel(x_cons, x_vals, x_obj, ei_c2v, ei_v2c, ei_v2o, ei_o2v, ei_c2o,
           ei_o2c, ea_c2v, ea_v2c, ea_v2o, ea_o2v, ea_c2o, ea_o2c, params):
    nc, nv, no = x_cons.shape[0], x_vals.shape[0], x_obj.shape[0]
    x = {
        "cons": _encode(x_cons, params["enc_cons"]),
        "vals": _encode(x_vals, params["enc_vals"]),
        "obj": _encode(x_obj, params["enc_obj"]),
    }
    ei = {"c2v": ei_c2v, "v2c": ei_v2c, "v2o": ei_v2o, "o2v": ei_o2v,
          "c2o": ei_c2o, "o2c": ei_o2c}
    ea = {"c2v": ea_c2v, "v2c": ea_v2c, "v2o": ea_v2o, "o2v": ea_o2v,
          "c2o": ea_c2o, "o2c": ea_o2c}
    topo = {"c2v": ("cons", "vals", nv), "v2c": ("vals", "cons", nc),
            "v2o": ("vals", "obj", no), "o2v": ("obj", "vals", nv),
            "c2o": ("cons", "obj", no), "o2c": ("obj", "cons", nc)}
    blk = {"vals": 2000, "cons": 2000, "obj": 1000}

    hiddens = []
    for i in range(len(params["gcns"])):
        L = params["gcns"][i]
        agg = {}
        for name in topo:
            src, _, n_dst = topo[name]
            agg[name] = _edge_agg(x[src], ei[name], ea[name],
                                  L[name]["edge"], n_dst)
        h2_vals, xn_vals = _post(x["vals"], agg["c2v"], agg["o2v"],
                                 L["c2v"], L["o2v"], blk["vals"])
        h2_cons, xn_cons = _post(x["cons"], agg["v2c"], agg["o2c"],
                                 L["v2c"], L["o2c"], blk["cons"])
        h2_obj, xn_obj = _post(x["obj"], agg["v2o"], agg["c2o"],
                               L["v2o"], L["c2o"], blk["obj"])
        hiddens.append((h2_cons, h2_vals))
        x = {"cons": xn_cons, "vals": xn_vals, "obj": xn_obj}

    vals = _pred(hiddens[0][1], hiddens[1][1], params["pred_vals"][0],
                 params["pred_vals"][1], blk["vals"])
    cons = _pred(hiddens[0][0], hiddens[1][0], params["pred_cons"][0],
                 params["pred_cons"][1], blk["cons"])
    return vals, cons
